# baseline (speedup 1.0000x reference)
import functools

import jax
import jax.numpy as jnp
from jax.experimental import pallas as pl
from jax.experimental.pallas import tpu as pltpu

EPS = 1e-5
CHUNK = 256
SUBT = 128
N_LAYERS = 4
D_MODEL = 1024
D_INNER = 2048
D_HEAD = 64
N_HEADS = 32
D_STATE = 64
D_CONV = 4
CONV_DIM = D_INNER + 2 * D_STATE
D_IN_PROJ = 2 * D_INNER + 2 * D_STATE + N_HEADS
SEQ = 2048
N_CHUNKS = SEQ // CHUNK
BATCH = 2

_BF = jnp.bfloat16
_F32 = jnp.float32


def _silu(v):
    return v * jax.nn.sigmoid(v)


def _lane_cumsum(v):
    n = v.shape[1]
    ii = jax.lax.broadcasted_iota(jnp.int32, v.shape, 1)
    k = 1
    while k < n:
        rolled = pltpu.roll(v, k, axis=1)
        v = v + jnp.where(ii < k, 0.0, rolled)
        k *= 2
    return v


def _mamba_kernel(x_ref, inw_ref, convw_ref, convb_ref, dtb_ref, alog_ref,
                  dexp_ref, gn_ref, outw_ref, rmsw_ref, e_ref, o_ref,
                  res_ref, state_ref, prevx_ref, y_ref):
    i = pl.program_id(1)
    c = pl.program_id(2)

    @pl.when(c == 0)
    def _():
        state_ref[...] = jnp.zeros_like(state_ref)
        prevx_ref[...] = jnp.zeros_like(prevx_ref)

    @pl.when(i == 0)
    def _():
        res_ref[pl.ds(c * CHUNK, CHUNK), :] = x_ref[0]

    res = res_ref[pl.ds(c * CHUNK, CHUNK), :]

    ms = jnp.mean(res * res, axis=-1, keepdims=True)
    xn = res * jax.lax.rsqrt(ms + EPS) * rmsw_ref[0]

    zx = jax.lax.dot_general(xn.astype(_BF), inw_ref[0],
                             (((1,), (1,)), ((), ())),
                             preferred_element_type=_F32)
    z = zx[:, :D_INNER]
    xbc_raw = zx[:, D_INNER:D_INNER + CONV_DIM]
    dtr = zx[:, D_INNER + CONV_DIM:]

    w = convw_ref[0]
    tail = prevx_ref[...]
    ii8 = jax.lax.broadcasted_iota(jnp.int32, (8, CONV_DIM), 0)
    xc_top = convb_ref[0] + xbc_raw[:8, :] * w[3:4, :]
    xc_rest = convb_ref[0] + xbc_raw[8:, :] * w[3:4, :]
    for s in (1, 2, 3):
        rc = pltpu.roll(xbc_raw, s, axis=0)
        rt = pltpu.roll(tail, s, axis=0)
        xc_top = xc_top + jnp.where(ii8 < s, rt, rc[:8, :]) * w[3 - s:4 - s, :]
        xc_rest = xc_rest + rc[8:, :] * w[3 - s:4 - s, :]
    prevx_ref[...] = xbc_raw[CHUNK - 8:, :]
    xbc = _silu(jnp.concatenate([xc_top, xc_rest], axis=0))

    x_in = xbc[:, :D_INNER]
    Bm = xbc[:, D_INNER:D_INNER + D_STATE]
    Cm = xbc[:, D_INNER + D_STATE:]
    B_bf = Bm.astype(_BF)
    C_bf = Cm.astype(_BF)

    dt = jax.nn.softplus(dtr + dtb_ref[0])
    A = -jnp.exp(alog_ref[0])
    dtA = dt * A
    dtAT = dtA.T

    E = e_ref[...]
    dt_exp = jnp.dot(dt.astype(_BF), E,
                     preferred_element_type=_F32)
    Xs_f = x_in * dt_exp
    Xs_bf = Xs_f.astype(_BF)

    ri = jax.lax.broadcasted_iota(jnp.int32, (SUBT, SUBT), 0)
    ci = jax.lax.broadcasted_iota(jnp.int32, (SUBT, SUBT), 1)
    trilm = ri >= ci

    for j in range(CHUNK // SUBT):
        r0 = j * SUBT
        locT = _lane_cumsum(dtAT[:, r0:r0 + SUBT])
        loc = locT.T
        last = loc[SUBT - 1:SUBT, :]
        Eloc = jnp.exp(loc)
        Edec = jnp.exp(last - loc)
        dlast = jnp.exp(last)
        Eloc_exp = jnp.dot(Eloc.astype(_BF), E,
                           preferred_element_type=_F32)
        Edec_exp = jnp.dot(Edec.astype(_BF), E,
                           preferred_element_type=_F32).astype(_BF)
        dlast_exp = jnp.dot(dlast.astype(_BF), E,
                            preferred_element_type=_F32)

        Bj = B_bf[r0:r0 + SUBT, :]
        Cj = C_bf[r0:r0 + SUBT, :]
        CB = jax.lax.dot_general(Cj, Bj, (((1,), (1,)), ((), ())),
                                 preferred_element_type=_F32)
        CBm = jnp.where(trilm, CB, 0.0)

        ST = state_ref[...]
        yo_all = jnp.dot(Cj, ST.astype(_BF),
                         preferred_element_type=_F32)
        Xsd_bf = Xs_bf[r0:r0 + SUBT, :] * Edec_exp

        for h in range(N_HEADS):
            acs_c = loc[:, h:h + 1]
            acs_r = locT[h:h + 1, :]
            dmat = jnp.minimum(acs_c - acs_r, 0.0)
            P = (CBm * jnp.exp(dmat)).astype(_BF)
            yd = jnp.dot(P, Xs_bf[r0:r0 + SUBT, h * D_HEAD:(h + 1) * D_HEAD],
                         preferred_element_type=_F32)
            y_ref[r0:r0 + SUBT, h * D_HEAD:(h + 1) * D_HEAD] = yd

        state_ref[...] = dlast_exp * ST + jax.lax.dot_general(
            Bj, Xsd_bf, (((0,), (0,)), ((), ())),
            preferred_element_type=_F32)

        ynew = (y_ref[r0:r0 + SUBT, :] + Eloc_exp * yo_all
                + dexp_ref[0] * x_in[r0:r0 + SUBT, :])
        u = ynew * _silu(z[r0:r0 + SUBT, :])
        ms2 = jnp.mean(u * u, axis=-1, keepdims=True)
        yn = u * jax.lax.rsqrt(ms2 + EPS) * gn_ref[0]
        out = jax.lax.dot_general(yn.astype(_BF), outw_ref[0],
                                  (((1,), (1,)), ((), ())),
                                  preferred_element_type=_F32)
        resn = res[r0:r0 + SUBT, :] + out
        res_ref[pl.ds(c * CHUNK + r0, SUBT), :] = resn
        o_ref[0, r0:r0 + SUBT, :] = resn


@functools.partial(jax.jit, static_argnames=("interpret",))
def kernel(x, in_proj_w, conv_w, conv_b, dt_bias, A_log, D, gnorm_w,
           out_proj_w, rms_w, interpret=False):
    inw_bf = in_proj_w.astype(_BF)
    outw_bf = out_proj_w.astype(_BF)
    convwT = jnp.transpose(conv_w, (0, 2, 1))
    convb = conv_b[:, None, :]
    dtb = dt_bias[:, None, :]
    alog = A_log[:, None, :]
    dexp = jnp.repeat(D, D_HEAD, axis=-1)[:, None, :]
    gn = gnorm_w[:, None, :]
    rw = rms_w[:, None, :]
    e_mat = jnp.repeat(jnp.eye(N_HEADS, dtype=_F32), D_HEAD,
                       axis=1).astype(_BF)

    grid = (BATCH, N_LAYERS, N_CHUNKS)
    return pl.pallas_call(
        _mamba_kernel,
        grid=grid,
        in_specs=[
            pl.BlockSpec((1, CHUNK, D_MODEL),
                         lambda b, i, c: (b, jnp.where(i == 0, c, 0), 0)),
            pl.BlockSpec((1, D_IN_PROJ, D_MODEL), lambda b, i, c: (i, 0, 0)),
            pl.BlockSpec((1, D_CONV, CONV_DIM), lambda b, i, c: (i, 0, 0)),
            pl.BlockSpec((1, 1, CONV_DIM), lambda b, i, c: (i, 0, 0)),
            pl.BlockSpec((1, 1, N_HEADS), lambda b, i, c: (i, 0, 0)),
            pl.BlockSpec((1, 1, N_HEADS), lambda b, i, c: (i, 0, 0)),
            pl.BlockSpec((1, 1, D_INNER), lambda b, i, c: (i, 0, 0)),
            pl.BlockSpec((1, 1, D_INNER), lambda b, i, c: (i, 0, 0)),
            pl.BlockSpec((1, D_MODEL, D_INNER), lambda b, i, c: (i, 0, 0)),
            pl.BlockSpec((1, 1, D_MODEL), lambda b, i, c: (i, 0, 0)),
            pl.BlockSpec((N_HEADS, D_INNER), lambda b, i, c: (0, 0)),
        ],
        out_specs=pl.BlockSpec((1, CHUNK, D_MODEL), lambda b, i, c: (b, c, 0)),
        out_shape=jax.ShapeDtypeStruct((BATCH, SEQ, D_MODEL), _F32),
        scratch_shapes=[
            pltpu.VMEM((SEQ, D_MODEL), _F32),
            pltpu.VMEM((D_STATE, D_INNER), _F32),
            pltpu.VMEM((8, CONV_DIM), _F32),
            pltpu.VMEM((CHUNK, D_INNER), _F32),
        ],
        compiler_params=pltpu.CompilerParams(
            dimension_semantics=("parallel", "arbitrary", "arbitrary"),
            vmem_limit_bytes=56 * 1024 * 1024,
        ),
        name="mamba2_stack",
        interpret=interpret,
    )(x, inw_bf, convwT, convb, dtb, alog, dexp, gn, outw_bf, rw, e_mat)


# speedup vs baseline: 1.0993x; 1.0993x over previous
import functools

import jax
import jax.numpy as jnp
from jax.experimental import pallas as pl
from jax.experimental.pallas import tpu as pltpu

EPS = 1e-5
CHUNK = 256
SUBT = 128
N_LAYERS = 4
D_MODEL = 1024
D_INNER = 2048
D_HEAD = 64
N_HEADS = 32
D_STATE = 64
D_CONV = 4
CONV_DIM = D_INNER + 2 * D_STATE
D_IN_PROJ = 2 * D_INNER + 2 * D_STATE + N_HEADS
SEQ = 2048
N_CHUNKS = SEQ // CHUNK
BATCH = 2

_BF = jnp.bfloat16
_F32 = jnp.float32


def _silu(v):
    return v * jax.nn.sigmoid(v)


def _lane_cumsum(v):
    n = v.shape[1]
    ii = jax.lax.broadcasted_iota(jnp.int32, v.shape, 1)
    k = 1
    while k < n:
        rolled = pltpu.roll(v, k, axis=1)
        v = v + jnp.where(ii < k, 0.0, rolled)
        k *= 2
    return v


def _mamba_kernel(x_ref, inw_ref, convw_ref, convb_ref, dtb_ref, alog_ref,
                  dexp_ref, gn_ref, outw_ref, rmsw_ref, e_ref, o_ref,
                  res_ref, state_ref, prevx_ref, y_ref):
    i = pl.program_id(1)
    c = pl.program_id(2)

    @pl.when(c == 0)
    def _():
        state_ref[...] = jnp.zeros_like(state_ref)
        prevx_ref[...] = jnp.zeros_like(prevx_ref)

    @pl.when(i == 0)
    def _():
        res_ref[pl.ds(c * CHUNK, CHUNK), :] = x_ref[0]

    res = res_ref[pl.ds(c * CHUNK, CHUNK), :]

    ms = jnp.mean(res * res, axis=-1, keepdims=True)
    xn = res * jax.lax.rsqrt(ms + EPS) * rmsw_ref[0]

    zx = jnp.dot(xn.astype(_BF), inw_ref[0],
                 preferred_element_type=_F32)
    z = zx[:, :D_INNER]
    xbc_raw = zx[:, D_INNER:D_INNER + CONV_DIM]
    dtr = zx[:, D_INNER + CONV_DIM:]

    w = convw_ref[0]
    tail = prevx_ref[...]
    ii8 = jax.lax.broadcasted_iota(jnp.int32, (8, CONV_DIM), 0)
    xc_top = convb_ref[0] + xbc_raw[:8, :] * w[3:4, :]
    xc_rest = convb_ref[0] + xbc_raw[8:, :] * w[3:4, :]
    for s in (1, 2, 3):
        rc = pltpu.roll(xbc_raw, s, axis=0)
        rt = pltpu.roll(tail, s, axis=0)
        xc_top = xc_top + jnp.where(ii8 < s, rt, rc[:8, :]) * w[3 - s:4 - s, :]
        xc_rest = xc_rest + rc[8:, :] * w[3 - s:4 - s, :]
    prevx_ref[...] = xbc_raw[CHUNK - 8:, :]
    xbc = _silu(jnp.concatenate([xc_top, xc_rest], axis=0))

    x_in = xbc[:, :D_INNER]
    Bm = xbc[:, D_INNER:D_INNER + D_STATE]
    Cm = xbc[:, D_INNER + D_STATE:]
    B_bf = Bm.astype(_BF)
    C_bf = Cm.astype(_BF)

    dt = jax.nn.softplus(dtr + dtb_ref[0])
    A = -jnp.exp(alog_ref[0])
    dtA = dt * A
    dtAT = dtA.T

    E = e_ref[...]
    dt_exp = jnp.dot(dt.astype(_BF), E,
                     preferred_element_type=_F32)
    Xs_f = x_in * dt_exp
    Xs_bf = Xs_f.astype(_BF)

    ri = jax.lax.broadcasted_iota(jnp.int32, (SUBT, SUBT), 0)
    ci = jax.lax.broadcasted_iota(jnp.int32, (SUBT, SUBT), 1)
    trilm = ri >= ci

    for j in range(CHUNK // SUBT):
        r0 = j * SUBT
        locT = _lane_cumsum(dtAT[:, r0:r0 + SUBT])
        loc = locT.T
        last = loc[SUBT - 1:SUBT, :]
        Eloc = jnp.exp(loc)
        Edec = jnp.exp(last - loc)
        dlast = jnp.exp(last)
        Eloc_exp = jnp.dot(Eloc.astype(_BF), E,
                           preferred_element_type=_F32)
        Edec_exp = jnp.dot(Edec.astype(_BF), E,
                           preferred_element_type=_F32).astype(_BF)
        dlast_exp = jnp.dot(dlast.astype(_BF), E,
                            preferred_element_type=_F32)

        Bj = B_bf[r0:r0 + SUBT, :]
        Cj = C_bf[r0:r0 + SUBT, :]
        CB = jax.lax.dot_general(Cj, Bj, (((1,), (1,)), ((), ())),
                                 preferred_element_type=_F32)
        CBm = jnp.where(trilm, CB, 0.0)

        ST = state_ref[...]
        yo_all = jnp.dot(Cj, ST.astype(_BF),
                         preferred_element_type=_F32)
        Xsd_bf = Xs_bf[r0:r0 + SUBT, :] * Edec_exp

        for h in range(N_HEADS):
            acs_c = loc[:, h:h + 1]
            acs_r = locT[h:h + 1, :]
            dmat = jnp.minimum(acs_c - acs_r, 0.0)
            P = (CBm * jnp.exp(dmat)).astype(_BF)
            yd = jnp.dot(P, Xs_bf[r0:r0 + SUBT, h * D_HEAD:(h + 1) * D_HEAD],
                         preferred_element_type=_F32)
            y_ref[r0:r0 + SUBT, h * D_HEAD:(h + 1) * D_HEAD] = yd

        state_ref[...] = dlast_exp * ST + jax.lax.dot_general(
            Bj, Xsd_bf, (((0,), (0,)), ((), ())),
            preferred_element_type=_F32)

        ynew = (y_ref[r0:r0 + SUBT, :] + Eloc_exp * yo_all
                + dexp_ref[0] * x_in[r0:r0 + SUBT, :])
        u = ynew * _silu(z[r0:r0 + SUBT, :])
        ms2 = jnp.mean(u * u, axis=-1, keepdims=True)
        yn = u * jax.lax.rsqrt(ms2 + EPS) * gn_ref[0]
        out = jnp.dot(yn.astype(_BF), outw_ref[0],
                      preferred_element_type=_F32)
        resn = res[r0:r0 + SUBT, :] + out
        res_ref[pl.ds(c * CHUNK + r0, SUBT), :] = resn
        o_ref[0, r0:r0 + SUBT, :] = resn


@functools.partial(jax.jit, static_argnames=("interpret",))
def kernel(x, in_proj_w, conv_w, conv_b, dt_bias, A_log, D, gnorm_w,
           out_proj_w, rms_w, interpret=False):
    inwT = jnp.transpose(in_proj_w.astype(_BF), (0, 2, 1))
    outwT = jnp.transpose(out_proj_w.astype(_BF), (0, 2, 1))
    convwT = jnp.transpose(conv_w, (0, 2, 1))
    convb = conv_b[:, None, :]
    dtb = dt_bias[:, None, :]
    alog = A_log[:, None, :]
    dexp = jnp.repeat(D, D_HEAD, axis=-1)[:, None, :]
    gn = gnorm_w[:, None, :]
    rw = rms_w[:, None, :]
    e_mat = jnp.repeat(jnp.eye(N_HEADS, dtype=_F32), D_HEAD,
                       axis=1).astype(_BF)

    grid = (BATCH, N_LAYERS, N_CHUNKS)
    return pl.pallas_call(
        _mamba_kernel,
        grid=grid,
        in_specs=[
            pl.BlockSpec((1, CHUNK, D_MODEL),
                         lambda b, i, c: (b, jnp.where(i == 0, c, 0), 0)),
            pl.BlockSpec((1, D_MODEL, D_IN_PROJ), lambda b, i, c: (i, 0, 0)),
            pl.BlockSpec((1, D_CONV, CONV_DIM), lambda b, i, c: (i, 0, 0)),
            pl.BlockSpec((1, 1, CONV_DIM), lambda b, i, c: (i, 0, 0)),
            pl.BlockSpec((1, 1, N_HEADS), lambda b, i, c: (i, 0, 0)),
            pl.BlockSpec((1, 1, N_HEADS), lambda b, i, c: (i, 0, 0)),
            pl.BlockSpec((1, 1, D_INNER), lambda b, i, c: (i, 0, 0)),
            pl.BlockSpec((1, 1, D_INNER), lambda b, i, c: (i, 0, 0)),
            pl.BlockSpec((1, D_INNER, D_MODEL), lambda b, i, c: (i, 0, 0)),
            pl.BlockSpec((1, 1, D_MODEL), lambda b, i, c: (i, 0, 0)),
            pl.BlockSpec((N_HEADS, D_INNER), lambda b, i, c: (0, 0)),
        ],
        out_specs=pl.BlockSpec((1, CHUNK, D_MODEL), lambda b, i, c: (b, c, 0)),
        out_shape=jax.ShapeDtypeStruct((BATCH, SEQ, D_MODEL), _F32),
        scratch_shapes=[
            pltpu.VMEM((SEQ, D_MODEL), _F32),
            pltpu.VMEM((D_STATE, D_INNER), _F32),
            pltpu.VMEM((8, CONV_DIM), _F32),
            pltpu.VMEM((CHUNK, D_INNER), _F32),
        ],
        compiler_params=pltpu.CompilerParams(
            dimension_semantics=("parallel", "arbitrary", "arbitrary"),
            vmem_limit_bytes=56 * 1024 * 1024,
        ),
        name="mamba2_stack",
        interpret=interpret,
    )(x, inwT, convwT, convb, dtb, alog, dexp, gn, outwT, rw, e_mat)


# speedup vs baseline: 1.1037x; 1.0040x over previous
import functools

import jax
import jax.numpy as jnp
from jax.experimental import pallas as pl
from jax.experimental.pallas import tpu as pltpu

EPS = 1e-5
CHUNK = 256
SUBT = 128
N_LAYERS = 4
D_MODEL = 1024
D_INNER = 2048
D_HEAD = 64
N_HEADS = 32
D_STATE = 64
D_CONV = 4
CONV_DIM = D_INNER + 2 * D_STATE
D_IN_PROJ = 2 * D_INNER + 2 * D_STATE + N_HEADS
SEQ = 2048
N_CHUNKS = SEQ // CHUNK
BATCH = 2

_BF = jnp.bfloat16
_F32 = jnp.float32


def _silu(v):
    return v * jax.nn.sigmoid(v)


def _lane_cumsum(v):
    n = v.shape[1]
    ii = jax.lax.broadcasted_iota(jnp.int32, v.shape, 1)
    k = 1
    while k < n:
        rolled = pltpu.roll(v, k, axis=1)
        v = v + jnp.where(ii < k, 0.0, rolled)
        k *= 2
    return v


def _mamba_kernel(x_ref, inw_ref, convw_ref, convb_ref, dtb_ref, alog_ref,
                  dexp_ref, gn_ref, outw_ref, rmsw_ref, e_ref, o_ref,
                  res_ref, state_ref, prevx_ref, y_ref):
    i = pl.program_id(1)
    c = pl.program_id(2)

    @pl.when(c == 0)
    def _():
        state_ref[...] = jnp.zeros_like(state_ref)
        prevx_ref[...] = jnp.zeros_like(prevx_ref)

    @pl.when(i == 0)
    def _():
        res_ref[pl.ds(c * CHUNK, CHUNK), :] = x_ref[0]

    res = res_ref[pl.ds(c * CHUNK, CHUNK), :]

    ms = jnp.mean(res * res, axis=-1, keepdims=True)
    xn = res * jax.lax.rsqrt(ms + EPS) * rmsw_ref[0]

    zx = jnp.dot(xn.astype(_BF), inw_ref[0],
                 preferred_element_type=_F32)
    z = zx[:, :D_INNER]
    xbc_raw = zx[:, D_INNER:D_INNER + CONV_DIM]
    dtr = zx[:, D_INNER + CONV_DIM:]

    w = convw_ref[0]
    tail = prevx_ref[...]
    ii8 = jax.lax.broadcasted_iota(jnp.int32, (8, CONV_DIM), 0)
    xc_top = convb_ref[0] + xbc_raw[:8, :] * w[3:4, :]
    xc_rest = convb_ref[0] + xbc_raw[8:, :] * w[3:4, :]
    for s in (1, 2, 3):
        rc = pltpu.roll(xbc_raw, s, axis=0)
        rt = pltpu.roll(tail, s, axis=0)
        xc_top = xc_top + jnp.where(ii8 < s, rt, rc[:8, :]) * w[3 - s:4 - s, :]
        xc_rest = xc_rest + rc[8:, :] * w[3 - s:4 - s, :]
    prevx_ref[...] = xbc_raw[CHUNK - 8:, :]
    xbc = _silu(jnp.concatenate([xc_top, xc_rest], axis=0))

    x_in = xbc[:, :D_INNER]
    Bm = xbc[:, D_INNER:D_INNER + D_STATE]
    Cm = xbc[:, D_INNER + D_STATE:]
    B_bf = Bm.astype(_BF)
    C_bf = Cm.astype(_BF)

    dt = jax.nn.softplus(dtr + dtb_ref[0])
    A = -jnp.exp(alog_ref[0])
    dtA = dt * A
    dtAT = dtA.T

    E = e_ref[...]
    dt_exp = jnp.dot(dt.astype(_BF), E,
                     preferred_element_type=_F32)
    Xs_f = x_in * dt_exp
    Xs_bf = Xs_f.astype(_BF)

    ri = jax.lax.broadcasted_iota(jnp.int32, (SUBT, SUBT), 0)
    ci = jax.lax.broadcasted_iota(jnp.int32, (SUBT, SUBT), 1)
    trilm = ri >= ci

    for j in range(CHUNK // SUBT):
        r0 = j * SUBT
        locT = _lane_cumsum(dtAT[:, r0:r0 + SUBT])
        loc = locT.T
        last = loc[SUBT - 1:SUBT, :]
        Eloc = jnp.exp(loc)
        Edec = jnp.exp(last - loc)
        dlast = jnp.exp(last)
        Eloc_exp = jnp.dot(Eloc.astype(_BF), E,
                           preferred_element_type=_F32)
        Edec_exp = jnp.dot(Edec.astype(_BF), E,
                           preferred_element_type=_F32).astype(_BF)
        dlast_exp = jnp.dot(dlast.astype(_BF), E,
                            preferred_element_type=_F32)

        Bj = B_bf[r0:r0 + SUBT, :]
        Cj = C_bf[r0:r0 + SUBT, :]
        CB = jax.lax.dot_general(Cj, Bj, (((1,), (1,)), ((), ())),
                                 preferred_element_type=_F32)
        CBm = jnp.where(trilm, CB, 0.0)

        ST = state_ref[...]
        yo_all = jnp.dot(Cj, ST.astype(_BF),
                         preferred_element_type=_F32)
        Xsd_bf = Xs_bf[r0:r0 + SUBT, :] * Edec_exp

        for h in range(N_HEADS):
            acs_c = loc[:, h:h + 1]
            acs_r = locT[h:h + 1, :]
            dmat = jnp.minimum(acs_c - acs_r, 0.0)
            P = (CBm * jnp.exp(dmat)).astype(_BF)
            yd = jnp.dot(P, Xs_bf[r0:r0 + SUBT, h * D_HEAD:(h + 1) * D_HEAD],
                         preferred_element_type=_F32)
            y_ref[r0:r0 + SUBT, h * D_HEAD:(h + 1) * D_HEAD] = yd

        state_ref[...] = dlast_exp * ST + jax.lax.dot_general(
            Bj, Xsd_bf, (((0,), (0,)), ((), ())),
            preferred_element_type=_F32)

        ynew = (y_ref[r0:r0 + SUBT, :] + Eloc_exp * yo_all
                + dexp_ref[0] * x_in[r0:r0 + SUBT, :])
        u = ynew * _silu(z[r0:r0 + SUBT, :])
        ms2 = jnp.mean(u * u, axis=-1, keepdims=True)
        yn = u * jax.lax.rsqrt(ms2 + EPS) * gn_ref[0]
        out = jnp.dot(yn.astype(_BF), outw_ref[0],
                      preferred_element_type=_F32)
        resn = res[r0:r0 + SUBT, :] + out
        res_ref[pl.ds(c * CHUNK + r0, SUBT), :] = resn
        o_ref[0, r0:r0 + SUBT, :] = resn


@functools.partial(jax.jit, static_argnames=("interpret",))
def kernel(x, in_proj_w, conv_w, conv_b, dt_bias, A_log, D, gnorm_w,
           out_proj_w, rms_w, interpret=False):
    inwT = jnp.transpose(in_proj_w, (0, 2, 1)).astype(_BF)
    outwT = jnp.transpose(out_proj_w, (0, 2, 1)).astype(_BF)
    convwT = jnp.transpose(conv_w, (0, 2, 1))
    convb = conv_b[:, None, :]
    dtb = dt_bias[:, None, :]
    alog = A_log[:, None, :]
    dexp = jnp.repeat(D, D_HEAD, axis=-1)[:, None, :]
    gn = gnorm_w[:, None, :]
    rw = rms_w[:, None, :]
    e_mat = jnp.repeat(jnp.eye(N_HEADS, dtype=_F32), D_HEAD,
                       axis=1).astype(_BF)

    grid = (BATCH, N_LAYERS, N_CHUNKS)
    return pl.pallas_call(
        _mamba_kernel,
        grid=grid,
        in_specs=[
            pl.BlockSpec((1, CHUNK, D_MODEL),
                         lambda b, i, c: (b, jnp.where(i == 0, c, 0), 0)),
            pl.BlockSpec((1, D_MODEL, D_IN_PROJ), lambda b, i, c: (i, 0, 0)),
            pl.BlockSpec((1, D_CONV, CONV_DIM), lambda b, i, c: (i, 0, 0)),
            pl.BlockSpec((1, 1, CONV_DIM), lambda b, i, c: (i, 0, 0)),
            pl.BlockSpec((1, 1, N_HEADS), lambda b, i, c: (i, 0, 0)),
            pl.BlockSpec((1, 1, N_HEADS), lambda b, i, c: (i, 0, 0)),
            pl.BlockSpec((1, 1, D_INNER), lambda b, i, c: (i, 0, 0)),
            pl.BlockSpec((1, 1, D_INNER), lambda b, i, c: (i, 0, 0)),
            pl.BlockSpec((1, D_INNER, D_MODEL), lambda b, i, c: (i, 0, 0)),
            pl.BlockSpec((1, 1, D_MODEL), lambda b, i, c: (i, 0, 0)),
            pl.BlockSpec((N_HEADS, D_INNER), lambda b, i, c: (0, 0)),
        ],
        out_specs=pl.BlockSpec((1, CHUNK, D_MODEL), lambda b, i, c: (b, c, 0)),
        out_shape=jax.ShapeDtypeStruct((BATCH, SEQ, D_MODEL), _F32),
        scratch_shapes=[
            pltpu.VMEM((SEQ, D_MODEL), _F32),
            pltpu.VMEM((D_STATE, D_INNER), _F32),
            pltpu.VMEM((8, CONV_DIM), _F32),
            pltpu.VMEM((CHUNK, D_INNER), _F32),
        ],
        compiler_params=pltpu.CompilerParams(
            dimension_semantics=("parallel", "arbitrary", "arbitrary"),
            vmem_limit_bytes=56 * 1024 * 1024,
        ),
        name="mamba2_stack",
        interpret=interpret,
    )(x, inwT, convwT, convb, dtb, alog, dexp, gn, outwT, rw, e_mat)


# speedup vs baseline: 1.1048x; 1.0010x over previous
import jax
import jax.numpy as jnp
from jax.experimental import pallas as pl
from jax.experimental.pallas import tpu as pltpu

EPS = 1e-5
CHUNK = 256
SUBT = 128
N_LAYERS = 4
D_MODEL = 1024
D_INNER = 2048
D_HEAD = 64
N_HEADS = 32
D_STATE = 64
D_CONV = 4
CONV_DIM = D_INNER + 2 * D_STATE
D_IN_PROJ = 2 * D_INNER + 2 * D_STATE + N_HEADS
SEQ = 2048
N_CHUNKS = SEQ // CHUNK
BATCH = 2

_BF = jnp.bfloat16
_F32 = jnp.float32


def _silu(v):
    return v * jax.nn.sigmoid(v)


def _lane_cumsum(v):
    n = v.shape[1]
    ii = jax.lax.broadcasted_iota(jnp.int32, v.shape, 1)
    k = 1
    while k < n:
        rolled = pltpu.roll(v, k, axis=1)
        v = v + jnp.where(ii < k, 0.0, rolled)
        k *= 2
    return v


def _mamba_kernel(x_ref, inw_ref, convw_ref, convb_ref, dtb_ref, alog_ref,
                  dexp_ref, gn_ref, outw_ref, rmsw_ref, e_ref, o_ref,
                  res_ref, state_ref, prevx_ref, y_ref):
    i = pl.program_id(1)
    c = pl.program_id(2)

    @pl.when(c == 0)
    def _():
        state_ref[...] = jnp.zeros_like(state_ref)
        prevx_ref[...] = jnp.zeros_like(prevx_ref)

    @pl.when(i == 0)
    def _():
        res_ref[pl.ds(c * CHUNK, CHUNK), :] = x_ref[0]

    res = res_ref[pl.ds(c * CHUNK, CHUNK), :]

    ms = jnp.mean(res * res, axis=-1, keepdims=True)
    xn = res * jax.lax.rsqrt(ms + EPS) * rmsw_ref[0]

    zx = jnp.dot(xn.astype(_BF), inw_ref[0],
                 preferred_element_type=_F32)
    z = zx[:, :D_INNER]
    xbc_raw = zx[:, D_INNER:D_INNER + CONV_DIM]
    dtr = zx[:, D_INNER + CONV_DIM:]

    w = convw_ref[0]
    tail = prevx_ref[...]
    ii8 = jax.lax.broadcasted_iota(jnp.int32, (8, CONV_DIM), 0)
    xc_top = convb_ref[0] + xbc_raw[:8, :] * w[3:4, :]
    xc_rest = convb_ref[0] + xbc_raw[8:, :] * w[3:4, :]
    for s in (1, 2, 3):
        rc = pltpu.roll(xbc_raw, s, axis=0)
        rt = pltpu.roll(tail, s, axis=0)
        xc_top = xc_top + jnp.where(ii8 < s, rt, rc[:8, :]) * w[3 - s:4 - s, :]
        xc_rest = xc_rest + rc[8:, :] * w[3 - s:4 - s, :]
    prevx_ref[...] = xbc_raw[CHUNK - 8:, :]
    xbc = _silu(jnp.concatenate([xc_top, xc_rest], axis=0))

    x_in = xbc[:, :D_INNER]
    Bm = xbc[:, D_INNER:D_INNER + D_STATE]
    Cm = xbc[:, D_INNER + D_STATE:]
    B_bf = Bm.astype(_BF)
    C_bf = Cm.astype(_BF)

    dt = jax.nn.softplus(dtr + dtb_ref[0])
    A = -jnp.exp(alog_ref[0])
    dtA = dt * A
    dtAT = dtA.T

    E = e_ref[...]
    dt_exp = jnp.dot(dt.astype(_BF), E,
                     preferred_element_type=_F32)
    Xs_f = x_in * dt_exp
    Xs_bf = Xs_f.astype(_BF)

    ri = jax.lax.broadcasted_iota(jnp.int32, (SUBT, SUBT), 0)
    ci = jax.lax.broadcasted_iota(jnp.int32, (SUBT, SUBT), 1)
    trilm = ri >= ci

    for j in range(CHUNK // SUBT):
        r0 = j * SUBT
        locT = _lane_cumsum(dtAT[:, r0:r0 + SUBT])
        loc = locT.T
        last = loc[SUBT - 1:SUBT, :]
        Eloc = jnp.exp(loc)
        Edec = jnp.exp(last - loc)
        dlast = jnp.exp(last)
        Eloc_exp = jnp.dot(Eloc.astype(_BF), E,
                           preferred_element_type=_F32)
        Edec_exp = jnp.dot(Edec.astype(_BF), E,
                           preferred_element_type=_F32).astype(_BF)
        dlast_exp = jnp.dot(dlast.astype(_BF), E,
                            preferred_element_type=_F32)

        Bj = B_bf[r0:r0 + SUBT, :]
        Cj = C_bf[r0:r0 + SUBT, :]
        CB = jax.lax.dot_general(Cj, Bj, (((1,), (1,)), ((), ())),
                                 preferred_element_type=_F32)
        CBm = jnp.where(trilm, CB, 0.0)

        ST = state_ref[...]
        yo_all = jnp.dot(Cj, ST.astype(_BF),
                         preferred_element_type=_F32)
        Xsd_bf = Xs_bf[r0:r0 + SUBT, :] * Edec_exp

        for h in range(N_HEADS):
            acs_c = loc[:, h:h + 1]
            acs_r = locT[h:h + 1, :]
            dmat = jnp.minimum(acs_c - acs_r, 0.0)
            P = (CBm * jnp.exp(dmat)).astype(_BF)
            yd = jnp.dot(P, Xs_bf[r0:r0 + SUBT, h * D_HEAD:(h + 1) * D_HEAD],
                         preferred_element_type=_F32)
            y_ref[r0:r0 + SUBT, h * D_HEAD:(h + 1) * D_HEAD] = yd

        state_ref[...] = dlast_exp * ST + jax.lax.dot_general(
            Bj, Xsd_bf, (((0,), (0,)), ((), ())),
            preferred_element_type=_F32)

        ynew = (y_ref[r0:r0 + SUBT, :] + Eloc_exp * yo_all
                + dexp_ref[0] * x_in[r0:r0 + SUBT, :])
        u = ynew * _silu(z[r0:r0 + SUBT, :])
        ms2 = jnp.mean(u * u, axis=-1, keepdims=True)
        yn = u * jax.lax.rsqrt(ms2 + EPS) * gn_ref[0]
        out = jnp.dot(yn.astype(_BF), outw_ref[0],
                      preferred_element_type=_F32)
        resn = res[r0:r0 + SUBT, :] + out
        res_ref[pl.ds(c * CHUNK + r0, SUBT), :] = resn
        o_ref[0, r0:r0 + SUBT, :] = resn


@jax.jit
def kernel(x, in_proj_w, conv_w, conv_b, dt_bias, A_log, D, gnorm_w,
           out_proj_w, rms_w):
    inwT = jnp.transpose(in_proj_w, (0, 2, 1)).astype(_BF)
    outwT = jnp.transpose(out_proj_w, (0, 2, 1)).astype(_BF)
    convwT = jnp.transpose(conv_w, (0, 2, 1))
    convb = conv_b[:, None, :]
    dtb = dt_bias[:, None, :]
    alog = A_log[:, None, :]
    dexp = jnp.repeat(D, D_HEAD, axis=-1)[:, None, :]
    gn = gnorm_w[:, None, :]
    rw = rms_w[:, None, :]
    e_mat = jnp.repeat(jnp.eye(N_HEADS, dtype=_F32), D_HEAD,
                       axis=1).astype(_BF)

    grid = (BATCH, N_LAYERS, N_CHUNKS)
    return pl.pallas_call(
        _mamba_kernel,
        grid=grid,
        in_specs=[
            pl.BlockSpec((1, CHUNK, D_MODEL),
                         lambda b, i, c: (b, jnp.where(i == 0, c, 0), 0)),
            pl.BlockSpec((1, D_MODEL, D_IN_PROJ), lambda b, i, c: (i, 0, 0)),
            pl.BlockSpec((1, D_CONV, CONV_DIM), lambda b, i, c: (i, 0, 0)),
            pl.BlockSpec((1, 1, CONV_DIM), lambda b, i, c: (i, 0, 0)),
            pl.BlockSpec((1, 1, N_HEADS), lambda b, i, c: (i, 0, 0)),
            pl.BlockSpec((1, 1, N_HEADS), lambda b, i, c: (i, 0, 0)),
            pl.BlockSpec((1, 1, D_INNER), lambda b, i, c: (i, 0, 0)),
            pl.BlockSpec((1, 1, D_INNER), lambda b, i, c: (i, 0, 0)),
            pl.BlockSpec((1, D_INNER, D_MODEL), lambda b, i, c: (i, 0, 0)),
            pl.BlockSpec((1, 1, D_MODEL), lambda b, i, c: (i, 0, 0)),
            pl.BlockSpec((N_HEADS, D_INNER), lambda b, i, c: (0, 0)),
        ],
        out_specs=pl.BlockSpec((1, CHUNK, D_MODEL), lambda b, i, c: (b, c, 0)),
        out_shape=jax.ShapeDtypeStruct((BATCH, SEQ, D_MODEL), _F32),
        scratch_shapes=[
            pltpu.VMEM((SEQ, D_MODEL), _F32),
            pltpu.VMEM((D_STATE, D_INNER), _F32),
            pltpu.VMEM((8, CONV_DIM), _F32),
            pltpu.VMEM((CHUNK, D_INNER), _F32),
        ],
        compiler_params=pltpu.CompilerParams(
            dimension_semantics=("parallel", "arbitrary", "arbitrary"),
            vmem_limit_bytes=56 * 1024 * 1024,
        ),
        name="mamba2_stack",
    )(x, inwT, convwT, convb, dtb, alog, dexp, gn, outwT, rw, e_mat)
